# baseline (speedup 1.0000x reference)
import functools

import jax
import jax.numpy as jnp
from jax import lax
from jax.experimental import pallas as pl
from jax.experimental.pallas import tpu as pltpu
from jax.experimental.pallas import tpu_sc as plsc

_NC = 2
_NS = 16
_L = 16
_NW = _NC * _NS


def _node_proj_body(feat_ref, w_ref, b_ref, t_ref):
    t_ref[...] = (
        jnp.dot(feat_ref[...], w_ref[...], preferred_element_type=jnp.float32)
        + b_ref[...]
    )


def _edge_mlp_body(vk2_ref, ef_ref, wek_ref, bek_ref, w1_ref, b1_ref,
                   w2_ref, b2_ref, out_ref):
    o = out_ref.shape[0]
    beh = vk2_ref.shape[0]
    ef = ef_ref[...]
    efc = jnp.concatenate([ef[:beh], ef[beh:]], axis=1)
    ekc = jnp.dot(efc, wek_ref[...], preferred_element_type=jnp.float32)
    vk = vk2_ref[...].astype(jnp.float32)
    a = jnp.maximum(vk + ekc + bek_ref[...], 0.0)
    a = jnp.maximum(
        jnp.dot(a, w1_ref[...], preferred_element_type=jnp.float32) + b1_ref[...], 0.0
    )
    at = lax.dot_general(
        w2_ref[...], a, (((0,), (1,)), ((), ())),
        preferred_element_type=jnp.float32,
    )
    at = jnp.maximum(at + b2_ref[...], 0.0)
    out_ref[:, :beh] = at[:o]
    out_ref[:, beh:] = at[o:]


def _make_gather_kernel(n_nodes, n_edges, h, beh, ch):
    nblk = n_edges // (2 * beh)
    ch2 = 2 * ch
    h2 = 2 * h
    rps = (n_nodes // _NS) // 8 * 8
    mesh = plsc.VectorSubcoreMesh(
        core_axis_name="c", subcore_axis_name="s", num_cores=_NC, num_subcores=_NS
    )

    @functools.partial(
        pl.kernel,
        out_type=jax.ShapeDtypeStruct((n_edges // 2 * h2,), jnp.bfloat16),
        mesh=mesh,
        scratch_types=[
            pltpu.VMEM_SHARED((n_nodes, h), jnp.float32),
            pltpu.VMEM_SHARED((n_nodes, h), jnp.float32),
            pltpu.VMEM((ch2,), jnp.int32),
            pltpu.VMEM((ch2,), jnp.int32),
            pltpu.VMEM((ch2,), jnp.int32),
            pltpu.VMEM((ch2,), jnp.int32),
            pltpu.VMEM((ch2, h), jnp.float32),
            pltpu.VMEM((ch2, h), jnp.float32),
            pltpu.VMEM((ch2, h), jnp.float32),
            pltpu.VMEM((ch2, h), jnp.float32),
            pltpu.VMEM((ch * h2,), jnp.bfloat16),
            pltpu.VMEM((ch * h2,), jnp.bfloat16),
            pltpu.SemaphoreType.DMA,
            pltpu.SemaphoreType.DMA,
            pltpu.SemaphoreType.DMA,
            pltpu.SemaphoreType.DMA,
            pltpu.SemaphoreType.DMA,
        ],
        compiler_params=pltpu.CompilerParams(
            use_tc_tiling_on_sc=False, needs_layout_passes=False
        ),
    )
    def gather_add(t_hbm, ei_hbm, out_hbm,
                   vsk_sh, vrk_sh, si0, di0, si1, di1,
                   rs0, rd0, rs1, rd1, ov0, ov1,
                   semi0, semi1, semg0, semg1, semo):
        cid = lax.axis_index("c")
        sid = lax.axis_index("s")
        wid = sid * _NC + cid
        wo = wid * ch

        for s in range(_NS):
            sz = rps if s < _NS - 1 else n_nodes - rps * (_NS - 1)

            @pl.when(sid == s)
            def _stage(s=s, sz=sz):
                pltpu.sync_copy(t_hbm.at[pl.ds(s * rps, sz), pl.ds(0, h)],
                                vsk_sh.at[pl.ds(s * rps, sz)])
                pltpu.sync_copy(t_hbm.at[pl.ds(s * rps, sz), pl.ds(h, h)],
                                vrk_sh.at[pl.ds(s * rps, sz)])

        plsc.subcore_barrier()

        def issue_idx(j, si, di, semi):
            lo = pl.multiple_of(j * 2 * beh + wo, 8)
            hi = pl.multiple_of(j * 2 * beh + beh + wo, 8)
            c0 = pltpu.async_copy(ei_hbm.at[pl.ds(lo, ch)],
                                  si.at[pl.ds(0, ch)], semi)
            c1 = pltpu.async_copy(ei_hbm.at[pl.ds(hi, ch)],
                                  si.at[pl.ds(ch, ch)], semi)
            c2 = pltpu.async_copy(ei_hbm.at[pl.ds(n_edges + lo, ch)],
                                  di.at[pl.ds(0, ch)], semi)
            c3 = pltpu.async_copy(ei_hbm.at[pl.ds(n_edges + hi, ch)],
                                  di.at[pl.ds(ch, ch)], semi)
            return c0, c1, c2, c3

        def drain_idx(si, di, semi):
            pltpu.make_async_copy(ei_hbm.at[pl.ds(0, ch)],
                                  si.at[pl.ds(0, ch)], semi).wait()
            pltpu.make_async_copy(ei_hbm.at[pl.ds(0, ch)],
                                  si.at[pl.ds(ch, ch)], semi).wait()
            pltpu.make_async_copy(ei_hbm.at[pl.ds(0, ch)],
                                  di.at[pl.ds(0, ch)], semi).wait()
            pltpu.make_async_copy(ei_hbm.at[pl.ds(0, ch)],
                                  di.at[pl.ds(ch, ch)], semi).wait()

        def issue_gather(si, di, rs, rd, semg):
            cs = pltpu.async_copy(vsk_sh.at[si], rs, semg)
            cd = pltpu.async_copy(vrk_sh.at[di], rd, semg)
            return cs, cd

        def combine(rs, rd, ov):
            @plsc.parallel_loop(0, ch, unroll=4)
            def add_rows(r):
                rb = r * h2
                for half_ofs, row in ((0, r), (h, ch + r)):
                    for c in range(h // (2 * _L)):
                        a = rs[row, pl.ds(2 * c * _L, _L)] + rd[row, pl.ds(2 * c * _L, _L)]
                        b = (rs[row, pl.ds((2 * c + 1) * _L, _L)]
                             + rd[row, pl.ds((2 * c + 1) * _L, _L)])
                        ov[pl.ds(rb + half_ofs + 2 * c * _L, 2 * _L)] = plsc.pack(
                            a, b, format=plsc.PackFormat.INTERLEAVED
                        )

        def flush(j, ov):
            oo = pl.multiple_of((j * beh + wo) * h2, 8)
            return pltpu.async_copy(ov, out_hbm.at[pl.ds(oo, ch * h2)], semo)

        issue_idx(0, si0, di0, semi0)
        issue_idx(1, si1, di1, semi1)

        def pair_body(i, carry):
            j0 = 2 * i
            j1 = 2 * i + 1
            drain_idx(si0, di0, semi0)
            g0 = issue_gather(si0, di0, rs0, rd0, semg0)
            drain_idx(si1, di1, semi1)
            g1 = issue_gather(si1, di1, rs1, rd1, semg1)
            g0[0].wait()
            g0[1].wait()

            @pl.when(j0 + 2 < nblk)
            def _pf0():
                issue_idx(j0 + 2, si0, di0, semi0)

            combine(rs0, rd0, ov0)
            o0 = flush(j0, ov0)
            g1[0].wait()
            g1[1].wait()

            @pl.when(j1 + 2 < nblk)
            def _pf1():
                issue_idx(j1 + 2, si1, di1, semi1)

            combine(rs1, rd1, ov1)
            o1 = flush(j1, ov1)
            o0.wait()
            o1.wait()
            return carry

        lax.fori_loop(0, nblk // 2, pair_body, 0)

        if nblk % 2:
            drain_idx(si0, di0, semi0)
            g = issue_gather(si0, di0, rs0, rd0, semg0)
            g[0].wait()
            g[1].wait()
            combine(rs0, rd0, ov0)
            flush(nblk - 1, ov0).wait()

    return gather_add


def _blkdiag(w):
    r, c = w.shape
    z = jnp.zeros((2 * r, 2 * c), w.dtype)
    return z.at[:r, :c].set(w).at[r:, c:].set(w)


def kernel(feat, efeat, edge_index, W_vsk, b_vsk, W_vrk, b_vrk, W_ek, b_ek, W1, b1,
           W2, b2):
    n, f_in = feat.shape
    e = efeat.shape[0]
    h = W_vsk.shape[0]
    o = W2.shape[0]
    half = e // 2
    beh = 1280
    ch = beh // _NW

    w_cat = jnp.concatenate([W_vsk.T, W_vrk.T], axis=1)
    b_cat = jnp.concatenate([b_vsk, b_vrk])[None, :]
    t_tab = pl.pallas_call(
        _node_proj_body,
        out_shape=jax.ShapeDtypeStruct((n, 2 * h), jnp.float32),
    )(feat, w_cat, b_cat)

    gather_add = _make_gather_kernel(n, e, h, beh, ch)
    vk2 = gather_add(t_tab, edge_index.reshape(-1)).reshape(half, 2 * h)

    perm = []
    for g in range(2 * h // 32):
        for i in range(16):
            perm.extend((g * 32 + i, g * 32 + 16 + i))
    perm = jnp.asarray(perm, jnp.int32)
    nblk = half // beh
    wek_d = _blkdiag(W_ek.T)[:, perm]
    bek_d = jnp.concatenate([b_ek, b_ek])[None, perm]
    w1_d = _blkdiag(W1.T)[perm, :]
    b1_d = jnp.concatenate([b1, b1])[None, :]
    w2_d = _blkdiag(W2.T)
    b2_d = jnp.concatenate([b2, b2])[:, None]
    out_t = pl.pallas_call(
        _edge_mlp_body,
        grid=(nblk,),
        in_specs=[
            pl.BlockSpec((beh, 2 * h), lambda i: (i, 0)),
            pl.BlockSpec((2 * beh, f_in), lambda i: (i, 0)),
            pl.BlockSpec((2 * f_in, 2 * h), lambda i: (0, 0)),
            pl.BlockSpec((1, 2 * h), lambda i: (0, 0)),
            pl.BlockSpec((2 * h, 2 * h), lambda i: (0, 0)),
            pl.BlockSpec((1, 2 * h), lambda i: (0, 0)),
            pl.BlockSpec((2 * h, 2 * o), lambda i: (0, 0)),
            pl.BlockSpec((2 * o, 1), lambda i: (0, 0)),
        ],
        out_specs=pl.BlockSpec((o, 2 * beh), lambda i: (0, i)),
        out_shape=jax.ShapeDtypeStruct((o, e), jnp.float32),
        compiler_params=pltpu.CompilerParams(
            dimension_semantics=("arbitrary",),
        ),
    )(vk2, efeat, wek_d, bek_d, w1_d, b1_d, w2_d, b2_d)
    return out_t.T


# speedup vs baseline: 1.4993x; 1.4993x over previous
import functools

import jax
import jax.numpy as jnp
from jax import lax
from jax.experimental import pallas as pl
from jax.experimental.pallas import tpu as pltpu
from jax.experimental.pallas import tpu_sc as plsc

_NC = 2
_NS = 16
_L = 16
_NW = _NC * _NS


def _node_proj_body(feat_ref, w_ref, b_ref, t_ref):
    t_ref[...] = (
        jnp.dot(feat_ref[...], w_ref[...], preferred_element_type=jnp.float32)
        + b_ref[...]
    )


def _edge_mlp_body(vk2_ref, ef_ref, wek_ref, bek_ref, w1_ref, b1_ref,
                   w2_ref, b2_ref, out_ref):
    o = out_ref.shape[0]
    beh = vk2_ref.shape[0]
    ef = ef_ref[...]
    efc = jnp.concatenate([ef[:beh], ef[beh:]], axis=1)
    ekc = jnp.dot(efc, wek_ref[...], preferred_element_type=jnp.float32)
    a = jnp.maximum(vk2_ref[...] + ekc + bek_ref[...], 0.0)
    a = jnp.maximum(
        jnp.dot(a, w1_ref[...], preferred_element_type=jnp.float32) + b1_ref[...], 0.0
    )
    at = lax.dot_general(
        w2_ref[...], a, (((0,), (1,)), ((), ())),
        preferred_element_type=jnp.float32,
    )
    at = jnp.maximum(at + b2_ref[...], 0.0)
    out_ref[:, :beh] = at[:o]
    out_ref[:, beh:] = at[o:]


def _make_gather_kernel(n_nodes, n_edges, h, beh, ch):
    nblk = n_edges // (2 * beh)
    ch2 = 2 * ch
    h2 = 2 * h
    rps = (n_nodes // _NS) // 8 * 8
    mesh = plsc.VectorSubcoreMesh(
        core_axis_name="c", subcore_axis_name="s", num_cores=_NC, num_subcores=_NS
    )

    @functools.partial(
        pl.kernel,
        out_type=jax.ShapeDtypeStruct((n_edges // 2 * h2,), jnp.float32),
        mesh=mesh,
        scratch_types=[
            pltpu.VMEM_SHARED((n_nodes, h), jnp.float32),
            pltpu.VMEM_SHARED((n_nodes, h), jnp.float32),
            pltpu.VMEM((ch2,), jnp.int32),
            pltpu.VMEM((ch2,), jnp.int32),
            pltpu.VMEM((ch2,), jnp.int32),
            pltpu.VMEM((ch2,), jnp.int32),
            pltpu.VMEM((ch2, h), jnp.float32),
            pltpu.VMEM((ch2, h), jnp.float32),
            pltpu.VMEM((ch2, h), jnp.float32),
            pltpu.VMEM((ch2, h), jnp.float32),
            pltpu.VMEM((ch * h2,), jnp.float32),
            pltpu.VMEM((ch * h2,), jnp.float32),
            pltpu.SemaphoreType.DMA,
            pltpu.SemaphoreType.DMA,
            pltpu.SemaphoreType.DMA,
            pltpu.SemaphoreType.DMA,
            pltpu.SemaphoreType.DMA,
        ],
        compiler_params=pltpu.CompilerParams(use_tc_tiling_on_sc=False),
    )
    def gather_add(t_hbm, ei_hbm, out_hbm,
                   vsk_sh, vrk_sh, si0, di0, si1, di1,
                   rs0, rd0, rs1, rd1, ov0, ov1,
                   semi0, semi1, semg0, semg1, semo):
        cid = lax.axis_index("c")
        sid = lax.axis_index("s")
        wid = sid * _NC + cid
        wo = wid * ch

        for s in range(_NS):
            sz = rps if s < _NS - 1 else n_nodes - rps * (_NS - 1)

            @pl.when(sid == s)
            def _stage(s=s, sz=sz):
                pltpu.sync_copy(t_hbm.at[pl.ds(s * rps, sz), pl.ds(0, h)],
                                vsk_sh.at[pl.ds(s * rps, sz)])
                pltpu.sync_copy(t_hbm.at[pl.ds(s * rps, sz), pl.ds(h, h)],
                                vrk_sh.at[pl.ds(s * rps, sz)])

        plsc.subcore_barrier()

        def issue_idx(j, si, di, semi):
            lo = pl.multiple_of(j * 2 * beh + wo, 8)
            hi = pl.multiple_of(j * 2 * beh + beh + wo, 8)
            c0 = pltpu.async_copy(ei_hbm.at[pl.ds(lo, ch)],
                                  si.at[pl.ds(0, ch)], semi)
            c1 = pltpu.async_copy(ei_hbm.at[pl.ds(hi, ch)],
                                  si.at[pl.ds(ch, ch)], semi)
            c2 = pltpu.async_copy(ei_hbm.at[pl.ds(n_edges + lo, ch)],
                                  di.at[pl.ds(0, ch)], semi)
            c3 = pltpu.async_copy(ei_hbm.at[pl.ds(n_edges + hi, ch)],
                                  di.at[pl.ds(ch, ch)], semi)
            return c0, c1, c2, c3

        def drain_idx(si, di, semi):
            pltpu.make_async_copy(ei_hbm.at[pl.ds(0, ch)],
                                  si.at[pl.ds(0, ch)], semi).wait()
            pltpu.make_async_copy(ei_hbm.at[pl.ds(0, ch)],
                                  si.at[pl.ds(ch, ch)], semi).wait()
            pltpu.make_async_copy(ei_hbm.at[pl.ds(0, ch)],
                                  di.at[pl.ds(0, ch)], semi).wait()
            pltpu.make_async_copy(ei_hbm.at[pl.ds(0, ch)],
                                  di.at[pl.ds(ch, ch)], semi).wait()

        def issue_gather(si, di, rs, rd, semg):
            cs = pltpu.async_copy(vsk_sh.at[si], rs, semg)
            cd = pltpu.async_copy(vrk_sh.at[di], rd, semg)
            return cs, cd

        def combine(rs, rd, ov):
            @plsc.parallel_loop(0, ch, unroll=4)
            def add_rows(r):
                rb = r * h2
                for c in range(h // _L):
                    sl = pl.ds(c * _L, _L)
                    ov[pl.ds(rb + c * _L, _L)] = rs[r, sl] + rd[r, sl]
                    ov[pl.ds(rb + h + c * _L, _L)] = rs[ch + r, sl] + rd[ch + r, sl]

        def flush(j, ov):
            oo = pl.multiple_of((j * beh + wo) * h2, 8)
            return pltpu.async_copy(ov, out_hbm.at[pl.ds(oo, ch * h2)], semo)

        issue_idx(0, si0, di0, semi0)
        issue_idx(1, si1, di1, semi1)

        def pair_body(i, carry):
            j0 = 2 * i
            j1 = 2 * i + 1
            drain_idx(si0, di0, semi0)
            g0 = issue_gather(si0, di0, rs0, rd0, semg0)
            drain_idx(si1, di1, semi1)
            g1 = issue_gather(si1, di1, rs1, rd1, semg1)
            g0[0].wait()
            g0[1].wait()

            @pl.when(j0 + 2 < nblk)
            def _pf0():
                issue_idx(j0 + 2, si0, di0, semi0)

            combine(rs0, rd0, ov0)
            o0 = flush(j0, ov0)
            g1[0].wait()
            g1[1].wait()

            @pl.when(j1 + 2 < nblk)
            def _pf1():
                issue_idx(j1 + 2, si1, di1, semi1)

            combine(rs1, rd1, ov1)
            o1 = flush(j1, ov1)
            o0.wait()
            o1.wait()
            return carry

        lax.fori_loop(0, nblk // 2, pair_body, 0)

        if nblk % 2:
            drain_idx(si0, di0, semi0)
            g = issue_gather(si0, di0, rs0, rd0, semg0)
            g[0].wait()
            g[1].wait()
            combine(rs0, rd0, ov0)
            flush(nblk - 1, ov0).wait()

    return gather_add


def _blkdiag(w):
    r, c = w.shape
    z = jnp.zeros((2 * r, 2 * c), w.dtype)
    return z.at[:r, :c].set(w).at[r:, c:].set(w)


def kernel(feat, efeat, edge_index, W_vsk, b_vsk, W_vrk, b_vrk, W_ek, b_ek, W1, b1,
           W2, b2):
    n, f_in = feat.shape
    e = efeat.shape[0]
    h = W_vsk.shape[0]
    o = W2.shape[0]
    half = e // 2
    beh = 1280
    ch = beh // _NW

    w_cat = jnp.concatenate([W_vsk.T, W_vrk.T], axis=1)
    b_cat = jnp.concatenate([b_vsk, b_vrk])[None, :]
    t_tab = pl.pallas_call(
        _node_proj_body,
        out_shape=jax.ShapeDtypeStruct((n, 2 * h), jnp.float32),
    )(feat, w_cat, b_cat)

    gather_add = _make_gather_kernel(n, e, h, beh, ch)
    vk2 = gather_add(t_tab, edge_index.reshape(-1)).reshape(half, 2 * h)

    nblk = half // beh
    wek_d = _blkdiag(W_ek.T)
    bek_d = jnp.concatenate([b_ek, b_ek])[None, :]
    w1_d = _blkdiag(W1.T)
    b1_d = jnp.concatenate([b1, b1])[None, :]
    w2_d = _blkdiag(W2.T)
    b2_d = jnp.concatenate([b2, b2])[:, None]
    out_t = pl.pallas_call(
        _edge_mlp_body,
        grid=(nblk,),
        in_specs=[
            pl.BlockSpec((beh, 2 * h), lambda i: (i, 0)),
            pl.BlockSpec((2 * beh, f_in), lambda i: (i, 0)),
            pl.BlockSpec((2 * f_in, 2 * h), lambda i: (0, 0)),
            pl.BlockSpec((1, 2 * h), lambda i: (0, 0)),
            pl.BlockSpec((2 * h, 2 * h), lambda i: (0, 0)),
            pl.BlockSpec((1, 2 * h), lambda i: (0, 0)),
            pl.BlockSpec((2 * h, 2 * o), lambda i: (0, 0)),
            pl.BlockSpec((2 * o, 1), lambda i: (0, 0)),
        ],
        out_specs=pl.BlockSpec((o, 2 * beh), lambda i: (0, i)),
        out_shape=jax.ShapeDtypeStruct((o, e), jnp.float32),
        compiler_params=pltpu.CompilerParams(
            dimension_semantics=("arbitrary",),
        ),
    )(vk2, efeat, wek_d, bek_d, w1_d, b1_d, w2_d, b2_d)
    return out_t.T


# speedup vs baseline: 1.8830x; 1.2559x over previous
import functools

import jax
import jax.numpy as jnp
from jax import lax
from jax.experimental import pallas as pl
from jax.experimental.pallas import tpu as pltpu
from jax.experimental.pallas import tpu_sc as plsc

_NC = 2
_NS = 16
_L = 16
_NW = _NC * _NS


def _node_proj_body(feat_ref, w_ref, b_ref, t_ref):
    t_ref[...] = (
        jnp.dot(feat_ref[...], w_ref[...], preferred_element_type=jnp.float32)
        + b_ref[...]
    )


def _edge_mlp_body(beh, vk2_ref, ef_ref, wek_ref, bek_ref, w1_ref, b1_ref,
                   w2_ref, b2_ref, out_ref):
    o = out_ref.shape[0]
    k = vk2_ref.shape[0] // beh
    ef = ef_ref[...]
    efc = jnp.concatenate(
        [
            jnp.concatenate(
                [ef[m * 2 * beh:m * 2 * beh + beh],
                 ef[m * 2 * beh + beh:(m + 1) * 2 * beh]],
                axis=1,
            )
            for m in range(k)
        ],
        axis=0,
    )
    ekc = jnp.dot(efc, wek_ref[...], preferred_element_type=jnp.float32)
    a = jnp.maximum(vk2_ref[...] + ekc + bek_ref[...], 0.0)
    a = jnp.maximum(
        jnp.dot(a, w1_ref[...], preferred_element_type=jnp.float32) + b1_ref[...], 0.0
    )
    at = lax.dot_general(
        w2_ref[...], a, (((0,), (1,)), ((), ())),
        preferred_element_type=jnp.float32,
    )
    at = jnp.maximum(at + b2_ref[...], 0.0)
    for m in range(k):
        out_ref[:, m * 2 * beh:m * 2 * beh + beh] = at[:o, m * beh:(m + 1) * beh]
        out_ref[:, m * 2 * beh + beh:(m + 1) * 2 * beh] = at[o:, m * beh:(m + 1) * beh]


def _make_gather_kernel(n_nodes, n_edges, h, beh, ch):
    nblk = n_edges // (2 * beh)
    ch2 = 2 * ch
    h2 = 2 * h
    rps = (n_nodes // _NS) // 8 * 8
    mesh = plsc.VectorSubcoreMesh(
        core_axis_name="c", subcore_axis_name="s", num_cores=_NC, num_subcores=_NS
    )

    @functools.partial(
        pl.kernel,
        out_type=jax.ShapeDtypeStruct((n_edges // 2 * h2,), jnp.float32),
        mesh=mesh,
        scratch_types=[
            pltpu.VMEM_SHARED((n_nodes, h), jnp.float32),
            pltpu.VMEM_SHARED((n_nodes, h), jnp.float32),
            pltpu.VMEM((ch2,), jnp.int32),
            pltpu.VMEM((ch2,), jnp.int32),
            pltpu.VMEM((ch2,), jnp.int32),
            pltpu.VMEM((ch2,), jnp.int32),
            pltpu.VMEM((ch2, h), jnp.float32),
            pltpu.VMEM((ch2, h), jnp.float32),
            pltpu.VMEM((ch2, h), jnp.float32),
            pltpu.VMEM((ch2, h), jnp.float32),
            pltpu.VMEM((ch * h2,), jnp.float32),
            pltpu.VMEM((ch * h2,), jnp.float32),
            pltpu.SemaphoreType.DMA,
            pltpu.SemaphoreType.DMA,
            pltpu.SemaphoreType.DMA,
            pltpu.SemaphoreType.DMA,
            pltpu.SemaphoreType.DMA,
        ],
        compiler_params=pltpu.CompilerParams(use_tc_tiling_on_sc=False),
    )
    def gather_add(t_hbm, ei_hbm, out_hbm,
                   vsk_sh, vrk_sh, si0, di0, si1, di1,
                   rs0, rd0, rs1, rd1, ov0, ov1,
                   semi0, semi1, semg0, semg1, semo):
        cid = lax.axis_index("c")
        sid = lax.axis_index("s")
        wid = sid * _NC + cid
        wo = wid * ch

        for s in range(_NS):
            sz = rps if s < _NS - 1 else n_nodes - rps * (_NS - 1)

            @pl.when(sid == s)
            def _stage(s=s, sz=sz):
                pltpu.sync_copy(t_hbm.at[pl.ds(s * rps, sz), pl.ds(0, h)],
                                vsk_sh.at[pl.ds(s * rps, sz)])
                pltpu.sync_copy(t_hbm.at[pl.ds(s * rps, sz), pl.ds(h, h)],
                                vrk_sh.at[pl.ds(s * rps, sz)])

        plsc.subcore_barrier()

        def issue_idx(j, si, di, semi):
            lo = pl.multiple_of(j * 2 * beh + wo, 8)
            hi = pl.multiple_of(j * 2 * beh + beh + wo, 8)
            c0 = pltpu.async_copy(ei_hbm.at[pl.ds(lo, ch)],
                                  si.at[pl.ds(0, ch)], semi)
            c1 = pltpu.async_copy(ei_hbm.at[pl.ds(hi, ch)],
                                  si.at[pl.ds(ch, ch)], semi)
            c2 = pltpu.async_copy(ei_hbm.at[pl.ds(n_edges + lo, ch)],
                                  di.at[pl.ds(0, ch)], semi)
            c3 = pltpu.async_copy(ei_hbm.at[pl.ds(n_edges + hi, ch)],
                                  di.at[pl.ds(ch, ch)], semi)
            return c0, c1, c2, c3

        def drain_idx(si, di, semi):
            pltpu.make_async_copy(ei_hbm.at[pl.ds(0, ch)],
                                  si.at[pl.ds(0, ch)], semi).wait()
            pltpu.make_async_copy(ei_hbm.at[pl.ds(0, ch)],
                                  si.at[pl.ds(ch, ch)], semi).wait()
            pltpu.make_async_copy(ei_hbm.at[pl.ds(0, ch)],
                                  di.at[pl.ds(0, ch)], semi).wait()
            pltpu.make_async_copy(ei_hbm.at[pl.ds(0, ch)],
                                  di.at[pl.ds(ch, ch)], semi).wait()

        def issue_gather(si, di, rs, rd, semg):
            cs = pltpu.async_copy(vsk_sh.at[si], rs, semg)
            cd = pltpu.async_copy(vrk_sh.at[di], rd, semg)
            return cs, cd

        def combine(rs, rd, ov):
            @plsc.parallel_loop(0, ch, unroll=4)
            def add_rows(r):
                rb = r * h2
                for c in range(h // _L):
                    sl = pl.ds(c * _L, _L)
                    ov[pl.ds(rb + c * _L, _L)] = rs[r, sl] + rd[r, sl]
                    ov[pl.ds(rb + h + c * _L, _L)] = rs[ch + r, sl] + rd[ch + r, sl]

        def flush(j, ov):
            oo = pl.multiple_of((j * beh + wo) * h2, 8)
            return pltpu.async_copy(ov, out_hbm.at[pl.ds(oo, ch * h2)], semo)

        issue_idx(0, si0, di0, semi0)
        issue_idx(1, si1, di1, semi1)

        def pair_body(i, carry):
            j0 = 2 * i
            j1 = 2 * i + 1
            drain_idx(si0, di0, semi0)
            g0 = issue_gather(si0, di0, rs0, rd0, semg0)
            drain_idx(si1, di1, semi1)
            g1 = issue_gather(si1, di1, rs1, rd1, semg1)
            g0[0].wait()
            g0[1].wait()

            @pl.when(j0 + 2 < nblk)
            def _pf0():
                issue_idx(j0 + 2, si0, di0, semi0)

            combine(rs0, rd0, ov0)
            o0 = flush(j0, ov0)
            g1[0].wait()
            g1[1].wait()

            @pl.when(j1 + 2 < nblk)
            def _pf1():
                issue_idx(j1 + 2, si1, di1, semi1)

            combine(rs1, rd1, ov1)
            o1 = flush(j1, ov1)
            o0.wait()
            o1.wait()
            return carry

        lax.fori_loop(0, nblk // 2, pair_body, 0)

        if nblk % 2:
            drain_idx(si0, di0, semi0)
            g = issue_gather(si0, di0, rs0, rd0, semg0)
            g[0].wait()
            g[1].wait()
            combine(rs0, rd0, ov0)
            flush(nblk - 1, ov0).wait()

    return gather_add


def _blkdiag(w):
    r, c = w.shape
    z = jnp.zeros((2 * r, 2 * c), w.dtype)
    return z.at[:r, :c].set(w).at[r:, c:].set(w)


def kernel(feat, efeat, edge_index, W_vsk, b_vsk, W_vrk, b_vrk, W_ek, b_ek, W1, b1,
           W2, b2):
    n, f_in = feat.shape
    e = efeat.shape[0]
    h = W_vsk.shape[0]
    o = W2.shape[0]
    half = e // 2
    beh = 1280
    ch = beh // _NW

    w_cat = jnp.concatenate([W_vsk.T, W_vrk.T], axis=1)
    b_cat = jnp.concatenate([b_vsk, b_vrk])[None, :]
    t_tab = pl.pallas_call(
        _node_proj_body,
        out_shape=jax.ShapeDtypeStruct((n, 2 * h), jnp.float32),
    )(feat, w_cat, b_cat)

    gather_add = _make_gather_kernel(n, e, h, beh, ch)
    vk2 = gather_add(t_tab, edge_index.reshape(-1)).reshape(half, 2 * h)

    nblk = half // beh
    wek_d = _blkdiag(W_ek.T)
    bek_d = jnp.concatenate([b_ek, b_ek])[None, :]
    w1_d = _blkdiag(W1.T)
    b1_d = jnp.concatenate([b1, b1])[None, :]
    w2_d = _blkdiag(W2.T)
    b2_d = jnp.concatenate([b2, b2])[:, None]
    k = 5
    out_t = pl.pallas_call(
        functools.partial(_edge_mlp_body, beh),
        grid=(nblk // k,),
        in_specs=[
            pl.BlockSpec((k * beh, 2 * h), lambda i: (i, 0)),
            pl.BlockSpec((k * 2 * beh, f_in), lambda i: (i, 0)),
            pl.BlockSpec((2 * f_in, 2 * h), lambda i: (0, 0)),
            pl.BlockSpec((1, 2 * h), lambda i: (0, 0)),
            pl.BlockSpec((2 * h, 2 * h), lambda i: (0, 0)),
            pl.BlockSpec((1, 2 * h), lambda i: (0, 0)),
            pl.BlockSpec((2 * h, 2 * o), lambda i: (0, 0)),
            pl.BlockSpec((2 * o, 1), lambda i: (0, 0)),
        ],
        out_specs=pl.BlockSpec((o, k * 2 * beh), lambda i: (0, i)),
        out_shape=jax.ShapeDtypeStruct((o, e), jnp.float32),
        compiler_params=pltpu.CompilerParams(
            dimension_semantics=("arbitrary",),
        ),
    )(vk2, efeat, wek_d, bek_d, w1_d, b1_d, w2_d, b2_d)
    return out_t.T


# speedup vs baseline: 1.9270x; 1.0234x over previous
import functools

import jax
import jax.numpy as jnp
from jax import lax
from jax.experimental import pallas as pl
from jax.experimental.pallas import tpu as pltpu
from jax.experimental.pallas import tpu_sc as plsc

_NC = 2
_NS = 16
_L = 16
_NW = _NC * _NS


def _node_proj_body(feat_ref, w_ref, b_ref, t_ref):
    t_ref[...] = (
        jnp.dot(feat_ref[...], w_ref[...], preferred_element_type=jnp.float32)
        + b_ref[...]
    )


def _edge_mlp_body(beh, vk2_ref, ef_ref, wek_ref, bek_ref, w1_ref, b1_ref,
                   w2_ref, b2_ref, out_ref):
    o = out_ref.shape[0]
    k = vk2_ref.shape[0] // beh
    ef = ef_ref[...]
    efc = jnp.concatenate(
        [
            jnp.concatenate(
                [ef[m * 2 * beh:m * 2 * beh + beh],
                 ef[m * 2 * beh + beh:(m + 1) * 2 * beh]],
                axis=1,
            )
            for m in range(k)
        ],
        axis=0,
    )
    ekc = jnp.dot(efc, wek_ref[...], preferred_element_type=jnp.float32)
    a = jnp.maximum(vk2_ref[...] + ekc + bek_ref[...], 0.0)
    a = jnp.maximum(
        jnp.dot(a, w1_ref[...], preferred_element_type=jnp.float32) + b1_ref[...], 0.0
    )
    at = lax.dot_general(
        w2_ref[...], a, (((0,), (1,)), ((), ())),
        preferred_element_type=jnp.float32,
    )
    at = jnp.maximum(at + b2_ref[...], 0.0)
    for m in range(k):
        out_ref[:, m * 2 * beh:m * 2 * beh + beh] = at[:o, m * beh:(m + 1) * beh]
        out_ref[:, m * 2 * beh + beh:(m + 1) * 2 * beh] = at[o:, m * beh:(m + 1) * beh]


def _make_gather_kernel(n_nodes, n_edges, h, beh, ch):
    nblk = n_edges // (2 * beh)
    ch2 = 2 * ch
    h2 = 2 * h
    rps = (n_nodes // _NS) // 8 * 8
    mesh = plsc.VectorSubcoreMesh(
        core_axis_name="c", subcore_axis_name="s", num_cores=_NC, num_subcores=_NS
    )

    @functools.partial(
        pl.kernel,
        out_type=jax.ShapeDtypeStruct((n_edges // 2 * h2,), jnp.float32),
        mesh=mesh,
        scratch_types=[
            pltpu.VMEM_SHARED((n_nodes, h), jnp.float32),
            pltpu.VMEM_SHARED((n_nodes, h), jnp.float32),
            pltpu.VMEM((ch2,), jnp.int32),
            pltpu.VMEM((ch2,), jnp.int32),
            pltpu.VMEM((ch2,), jnp.int32),
            pltpu.VMEM((ch2,), jnp.int32),
            pltpu.VMEM((ch2, h), jnp.float32),
            pltpu.VMEM((ch2, h), jnp.float32),
            pltpu.VMEM((ch2, h), jnp.float32),
            pltpu.VMEM((ch2, h), jnp.float32),
            pltpu.VMEM((ch * h2,), jnp.float32),
            pltpu.VMEM((ch * h2,), jnp.float32),
            pltpu.SemaphoreType.DMA,
            pltpu.SemaphoreType.DMA,
            pltpu.SemaphoreType.DMA,
            pltpu.SemaphoreType.DMA,
            pltpu.SemaphoreType.DMA,
        ],
        compiler_params=pltpu.CompilerParams(use_tc_tiling_on_sc=False),
    )
    def gather_add(t_hbm, ei_hbm, out_hbm,
                   vsk_sh, vrk_sh, si0, di0, si1, di1,
                   rs0, rd0, rs1, rd1, ov0, ov1,
                   semi0, semi1, semg0, semg1, semo):
        cid = lax.axis_index("c")
        sid = lax.axis_index("s")
        wid = sid * _NC + cid
        wo = wid * ch

        for s in range(_NS):
            sz = rps if s < _NS - 1 else n_nodes - rps * (_NS - 1)

            @pl.when(sid == s)
            def _stage(s=s, sz=sz):
                pltpu.sync_copy(t_hbm.at[pl.ds(s * rps, sz), pl.ds(0, h)],
                                vsk_sh.at[pl.ds(s * rps, sz)])
                pltpu.sync_copy(t_hbm.at[pl.ds(s * rps, sz), pl.ds(h, h)],
                                vrk_sh.at[pl.ds(s * rps, sz)])

        plsc.subcore_barrier()

        def issue_idx(j, si, di, semi):
            lo = pl.multiple_of(j * 2 * beh + wo, 8)
            hi = pl.multiple_of(j * 2 * beh + beh + wo, 8)
            c0 = pltpu.async_copy(ei_hbm.at[pl.ds(lo, ch)],
                                  si.at[pl.ds(0, ch)], semi)
            c1 = pltpu.async_copy(ei_hbm.at[pl.ds(hi, ch)],
                                  si.at[pl.ds(ch, ch)], semi)
            c2 = pltpu.async_copy(ei_hbm.at[pl.ds(n_edges + lo, ch)],
                                  di.at[pl.ds(0, ch)], semi)
            c3 = pltpu.async_copy(ei_hbm.at[pl.ds(n_edges + hi, ch)],
                                  di.at[pl.ds(ch, ch)], semi)
            return c0, c1, c2, c3

        def drain_idx(si, di, semi):
            pltpu.make_async_copy(ei_hbm.at[pl.ds(0, ch)],
                                  si.at[pl.ds(0, ch)], semi).wait()
            pltpu.make_async_copy(ei_hbm.at[pl.ds(0, ch)],
                                  si.at[pl.ds(ch, ch)], semi).wait()
            pltpu.make_async_copy(ei_hbm.at[pl.ds(0, ch)],
                                  di.at[pl.ds(0, ch)], semi).wait()
            pltpu.make_async_copy(ei_hbm.at[pl.ds(0, ch)],
                                  di.at[pl.ds(ch, ch)], semi).wait()

        def issue_gather(si, di, rs, rd, semg):
            cs = pltpu.async_copy(vsk_sh.at[si], rs, semg)
            cd = pltpu.async_copy(vrk_sh.at[di], rd, semg)
            return cs, cd

        def combine(rs, rd, ov):
            @plsc.parallel_loop(0, ch, unroll=8)
            def add_rows(r):
                rb = r * h2
                for c in range(h // _L):
                    sl = pl.ds(c * _L, _L)
                    ov[pl.ds(rb + c * _L, _L)] = rs[r, sl] + rd[r, sl]
                    ov[pl.ds(rb + h + c * _L, _L)] = rs[ch + r, sl] + rd[ch + r, sl]

        def flush(j, ov):
            oo = pl.multiple_of((j * beh + wo) * h2, 8)
            return pltpu.async_copy(ov, out_hbm.at[pl.ds(oo, ch * h2)], semo)

        def drain_flush(ov):
            pltpu.make_async_copy(out_hbm.at[pl.ds(0, ch * h2)], ov, semo).wait()

        issue_idx(0, si0, di0, semi0)
        issue_idx(1, si1, di1, semi1)

        def pair_body(i, carry):
            j0 = 2 * i
            j1 = 2 * i + 1
            drain_idx(si0, di0, semi0)
            g0 = issue_gather(si0, di0, rs0, rd0, semg0)
            drain_idx(si1, di1, semi1)
            g1 = issue_gather(si1, di1, rs1, rd1, semg1)
            g0[0].wait()
            g0[1].wait()

            @pl.when(j0 + 2 < nblk)
            def _pf0():
                issue_idx(j0 + 2, si0, di0, semi0)

            @pl.when(i > 0)
            def _dr0():
                drain_flush(ov0)

            combine(rs0, rd0, ov0)
            flush(j0, ov0)
            g1[0].wait()
            g1[1].wait()

            @pl.when(j1 + 2 < nblk)
            def _pf1():
                issue_idx(j1 + 2, si1, di1, semi1)

            @pl.when(i > 0)
            def _dr1():
                drain_flush(ov1)

            combine(rs1, rd1, ov1)
            flush(j1, ov1)
            return carry

        lax.fori_loop(0, nblk // 2, pair_body, 0)
        drain_flush(ov0)
        drain_flush(ov1)

        if nblk % 2:
            drain_idx(si0, di0, semi0)
            g = issue_gather(si0, di0, rs0, rd0, semg0)
            g[0].wait()
            g[1].wait()
            combine(rs0, rd0, ov0)
            flush(nblk - 1, ov0).wait()

    return gather_add


def _blkdiag(w):
    r, c = w.shape
    z = jnp.zeros((2 * r, 2 * c), w.dtype)
    return z.at[:r, :c].set(w).at[r:, c:].set(w)


def kernel(feat, efeat, edge_index, W_vsk, b_vsk, W_vrk, b_vrk, W_ek, b_ek, W1, b1,
           W2, b2):
    n, f_in = feat.shape
    e = efeat.shape[0]
    h = W_vsk.shape[0]
    o = W2.shape[0]
    half = e // 2
    beh = 1280
    ch = beh // _NW

    w_cat = jnp.concatenate([W_vsk.T, W_vrk.T], axis=1)
    b_cat = jnp.concatenate([b_vsk, b_vrk])[None, :]
    t_tab = pl.pallas_call(
        _node_proj_body,
        out_shape=jax.ShapeDtypeStruct((n, 2 * h), jnp.float32),
    )(feat, w_cat, b_cat)

    gather_add = _make_gather_kernel(n, e, h, beh, ch)
    vk2 = gather_add(t_tab, edge_index.reshape(-1)).reshape(half, 2 * h)

    nblk = half // beh
    wek_d = _blkdiag(W_ek.T)
    bek_d = jnp.concatenate([b_ek, b_ek])[None, :]
    w1_d = _blkdiag(W1.T)
    b1_d = jnp.concatenate([b1, b1])[None, :]
    w2_d = _blkdiag(W2.T)
    b2_d = jnp.concatenate([b2, b2])[:, None]
    k = 5
    out_t = pl.pallas_call(
        functools.partial(_edge_mlp_body, beh),
        grid=(nblk // k,),
        in_specs=[
            pl.BlockSpec((k * beh, 2 * h), lambda i: (i, 0)),
            pl.BlockSpec((k * 2 * beh, f_in), lambda i: (i, 0)),
            pl.BlockSpec((2 * f_in, 2 * h), lambda i: (0, 0)),
            pl.BlockSpec((1, 2 * h), lambda i: (0, 0)),
            pl.BlockSpec((2 * h, 2 * h), lambda i: (0, 0)),
            pl.BlockSpec((1, 2 * h), lambda i: (0, 0)),
            pl.BlockSpec((2 * h, 2 * o), lambda i: (0, 0)),
            pl.BlockSpec((2 * o, 1), lambda i: (0, 0)),
        ],
        out_specs=pl.BlockSpec((o, k * 2 * beh), lambda i: (0, i)),
        out_shape=jax.ShapeDtypeStruct((o, e), jnp.float32),
        compiler_params=pltpu.CompilerParams(
            dimension_semantics=("arbitrary",),
        ),
    )(vk2, efeat, wek_d, bek_d, w1_d, b1_d, w2_d, b2_d)
    return out_t.T
